# baseline (speedup 1.0000x reference)
import functools

import jax
import jax.numpy as jnp
from jax import lax
from jax.experimental import pallas as pl
from jax.experimental.pallas import tpu as pltpu
from jax.experimental.pallas import tpu_sc as plsc

D = 64
SCALE = 8.0
NC, NS = 2, 16
NW = NC * NS
B = 4096 * 200
CHUNK = 128
NCH = B // (NW * CHUNK)
RPW = B // NW
NBUF = 4


def _body(idx_hbm, table_hbm, out_hbm, idx_v, mult_v, d0, d1, d2, d3,
          g0, g1, g2, g3, o0, o1, o2, o3):
    data = (d0, d1, d2, d3)
    gsem = (g0, g1, g2, g3)
    osem = (o0, o1, o2, o3)
    c = lax.axis_index("c")
    s = lax.axis_index("s")
    w = c * NS + s
    row0 = w * RPW

    pltpu.sync_copy(idx_hbm.at[w], idx_v)

    def start_gather(j, b):
        pltpu.async_copy(table_hbm.at[idx_v.at[j]], data[b], gsem[b])

    def wait_gather(j, b):
        pltpu.make_async_copy(table_hbm.at[idx_v.at[j]], data[b],
                              gsem[b]).wait()

    def start_out(j, b):
        pltpu.async_copy(data[b],
                         out_hbm.at[pl.ds(row0 + j * CHUNK, CHUNK)], osem[b])

    def wait_out(j, b):
        pltpu.make_async_copy(data[b],
                              out_hbm.at[pl.ds(row0 + j * CHUNK, CHUNK)],
                              osem[b]).wait()

    def compute(j, b):
        dref = data[b]

        def mkmult(gi, carry):
            sl = pl.ds(gi * 16, 16)
            idxv = idx_v[j, sl]
            mult_v[sl] = jnp.where(idxv != 0,
                                   jnp.float32(SCALE), jnp.float32(0.0))
            return carry

        lax.fori_loop(0, CHUNK // 16, mkmult, 0)

        def rowloop(r, carry):
            msp = plsc.load_gather(mult_v, [jnp.full((16,), r, jnp.int32)])
            for q in range(4):
                sl = pl.ds(q * 16, 16)
                dref[r, sl] = dref[r, sl] * msp
            return carry

        lax.fori_loop(0, CHUNK, rowloop, 0)

    for b in range(NBUF - 1):
        start_gather(b, b)

    def outer(g, carry):
        for b in range(NBUF):
            j = g * NBUF + b
            wait_gather(j, b)
            compute(j, b)
            start_out(j, b)
            bb = (b + NBUF - 1) % NBUF
            nj = j + NBUF - 1

            @pl.when(j >= 1)
            def _():
                wait_out(j - 1, bb)

            @pl.when(nj < NCH)
            def _():
                start_gather(nj, bb)

        return carry

    lax.fori_loop(0, NCH // NBUF, outer, 0)
    wait_out(NCH - 1, (NCH - 1) % NBUF)


@jax.jit
def _run(idx3, table):
    mesh = plsc.VectorSubcoreMesh(core_axis_name="c", subcore_axis_name="s")
    f = pl.kernel(
        _body,
        out_type=jax.ShapeDtypeStruct((B, D), jnp.float32),
        mesh=mesh,
        compiler_params=pltpu.CompilerParams(needs_layout_passes=False,
                                             use_tc_tiling_on_sc=False),
        scratch_types=(
            [pltpu.VMEM((NCH, CHUNK), jnp.int32),
             pltpu.VMEM((CHUNK,), jnp.float32)]
            + [pltpu.VMEM((CHUNK, D), jnp.float32)] * NBUF
            + [pltpu.SemaphoreType.DMA] * (2 * NBUF)
        ),
    )
    return f(idx3, table)


def kernel(input, lookup_table):
    ids = input.astype(jnp.int32).reshape(NW, NCH, CHUNK)
    out = _run(ids, lookup_table)
    return out.reshape(input.shape[0], input.shape[1], D)
